# Initial kernel scaffold; baseline (speedup 1.0000x reference)
import functools

import jax
import jax.numpy as jnp
from jax import lax
from jax.experimental import pallas as pl
from jax.experimental.pallas import tpu as pltpu
from jax.experimental.pallas import tpu_sc as plsc

N_BUCKETS = 1024
N_POINTS = 8388608

NC = 2
NS = 16
NW = NC * NS
LANES = 16

PPW = N_POINTS // NW
CHUNK = 16384
N_CHUNKS = PPW // CHUNK


def _table_body(logits_ref, lo_ref, hi_ref, out_ref):
    l = logits_ref[...]
    m = jnp.max(l)
    lse = jnp.log(jnp.sum(jnp.exp(l - m))) + m
    w = hi_ref[...] - lo_ref[...]
    out_ref[...] = (l - lse) - jnp.log(w)


def _build_table(logits, boundaries):
    lo = boundaries[:-1].reshape(8, 128)
    hi = boundaries[1:].reshape(8, 128)
    table = pl.pallas_call(
        _table_body,
        out_shape=jax.ShapeDtypeStruct((8, 128), jnp.float32),
    )(logits.reshape(8, 128), lo, hi)
    return table.reshape(N_BUCKETS)


def _sc_body(table_hbm, y_hbm, out_hbm, table_v, y_v, out_v):
    wid = lax.axis_index("s") * NC + lax.axis_index("c")
    base = wid * PPW
    pltpu.sync_copy(table_hbm, table_v)

    def chunk_body(c, _):
        off = base + c * CHUNK
        pltpu.sync_copy(y_hbm.at[pl.ds(off, CHUNK)], y_v)

        def grp_body(i, _):
            s = i * LANES
            y16 = y_v[pl.ds(s, LANES)]
            idx = jnp.minimum((y16 * float(N_BUCKETS)).astype(jnp.int32),
                              N_BUCKETS - 1)
            out_v[pl.ds(s, LANES)] = plsc.load_gather(table_v, [idx])
            return 0

        lax.fori_loop(0, CHUNK // LANES, grp_body, 0)
        pltpu.sync_copy(out_v, out_hbm.at[pl.ds(off, CHUNK)])
        return 0

    lax.fori_loop(0, N_CHUNKS, chunk_body, 0)


@jax.jit
def _sc_gather(table, y):
    mesh = plsc.VectorSubcoreMesh(core_axis_name="c", subcore_axis_name="s")
    return pl.kernel(
        _sc_body,
        out_type=jax.ShapeDtypeStruct((N_POINTS,), jnp.float32),
        mesh=mesh,
        scratch_types=[
            pltpu.VMEM((N_BUCKETS,), jnp.float32),
            pltpu.VMEM((CHUNK,), jnp.float32),
            pltpu.VMEM((CHUNK,), jnp.float32),
        ],
    )(table, y)


def kernel(logits, y, boundaries):
    table = _build_table(logits, boundaries)
    return _sc_gather(table, y)


# speedup vs baseline: 7417.2562x; 7417.2562x over previous
import functools

import jax
import jax.numpy as jnp
from jax import lax
from jax.experimental import pallas as pl
from jax.experimental.pallas import tpu as pltpu
from jax.experimental.pallas import tpu_sc as plsc

N_BUCKETS = 1024
N_POINTS = 8388608

NC = 2
NS = 16
NW = NC * NS
LANES = 16

PPW = N_POINTS // NW
CHUNK = 16384
N_CHUNKS = PPW // CHUNK


def _table_body(logits_ref, lo_ref, hi_ref, out_ref):
    l = logits_ref[...]
    m = jnp.max(l)
    lse = jnp.log(jnp.sum(jnp.exp(l - m))) + m
    w = hi_ref[...] - lo_ref[...]
    out_ref[...] = (l - lse) - jnp.log(w)


def _build_table(logits, boundaries):
    lo = boundaries[:-1].reshape(8, 128)
    hi = boundaries[1:].reshape(8, 128)
    table = pl.pallas_call(
        _table_body,
        out_shape=jax.ShapeDtypeStruct((8, 128), jnp.float32),
    )(logits.reshape(8, 128), lo, hi)
    return table.reshape(N_BUCKETS)


def _sc_body(table_hbm, y_hbm, out_hbm, table_v, y_v, out_v):
    wid = lax.axis_index("s") * NC + lax.axis_index("c")
    base = wid * PPW
    pltpu.sync_copy(table_hbm, table_v)

    def chunk_body(c, _):
        off = base + c * CHUNK
        pltpu.sync_copy(y_hbm.at[pl.ds(off, CHUNK)], y_v)

        def grp_body(i, _):
            s = i * LANES
            y16 = y_v[pl.ds(s, LANES)]
            idx = jnp.minimum((y16 * float(N_BUCKETS)).astype(jnp.int32),
                              N_BUCKETS - 1)
            out_v[pl.ds(s, LANES)] = plsc.load_gather(table_v, [idx])
            return 0

        lax.fori_loop(0, CHUNK // LANES, grp_body, 0)
        pltpu.sync_copy(out_v, out_hbm.at[pl.ds(off, CHUNK)])
        return 0

    lax.fori_loop(0, N_CHUNKS, chunk_body, 0)


@jax.jit
def _sc_gather(table, y):
    mesh = plsc.VectorSubcoreMesh(core_axis_name="c", subcore_axis_name="s")
    return pl.kernel(
        _sc_body,
        out_type=jax.ShapeDtypeStruct((N_POINTS,), jnp.float32),
        mesh=mesh,
        compiler_params=pltpu.CompilerParams(needs_layout_passes=False),
        scratch_types=[
            pltpu.VMEM((N_BUCKETS,), jnp.float32),
            pltpu.VMEM((CHUNK,), jnp.float32),
            pltpu.VMEM((CHUNK,), jnp.float32),
        ],
    )(table, y)


def kernel(logits, y, boundaries):
    table = _build_table(logits, boundaries)
    return _sc_gather(table, y)


# speedup vs baseline: 12125.2121x; 1.6347x over previous
import functools

import jax
import jax.numpy as jnp
from jax import lax
from jax.experimental import pallas as pl
from jax.experimental.pallas import tpu as pltpu
from jax.experimental.pallas import tpu_sc as plsc

N_BUCKETS = 1024
N_POINTS = 8388608

NC = 2
NS = 16
NW = NC * NS
LANES = 16

PPW = N_POINTS // NW
CHUNK = 16384
N_CHUNKS = PPW // CHUNK


def _table_body(logits_ref, lo_ref, hi_ref, out_ref):
    l = logits_ref[...]
    m = jnp.max(l)
    lse = jnp.log(jnp.sum(jnp.exp(l - m))) + m
    w = hi_ref[...] - lo_ref[...]
    out_ref[...] = (l - lse) - jnp.log(w)


def _build_table(logits, boundaries):
    lo = boundaries[:-1].reshape(8, 128)
    hi = boundaries[1:].reshape(8, 128)
    table = pl.pallas_call(
        _table_body,
        out_shape=jax.ShapeDtypeStruct((8, 128), jnp.float32),
    )(logits.reshape(8, 128), lo, hi)
    return table.reshape(N_BUCKETS)


def _sc_body(table_hbm, y_hbm, out_hbm, table_v, y_v, out_v):
    wid = lax.axis_index("s") * NC + lax.axis_index("c")
    base = wid * PPW
    pltpu.sync_copy(table_hbm, table_v)

    def chunk_body(c, _):
        off = base + c * CHUNK
        pltpu.sync_copy(y_hbm.at[pl.ds(off, CHUNK)], y_v)

        @plsc.parallel_loop(0, CHUNK // LANES, unroll=8)
        def grp_body(i):
            s = i * LANES
            y16 = y_v[pl.ds(s, LANES)]
            idx = jnp.minimum((y16 * float(N_BUCKETS)).astype(jnp.int32),
                              N_BUCKETS - 1)
            out_v[pl.ds(s, LANES)] = plsc.load_gather(table_v, [idx])
        pltpu.sync_copy(out_v, out_hbm.at[pl.ds(off, CHUNK)])
        return 0

    lax.fori_loop(0, N_CHUNKS, chunk_body, 0)


@jax.jit
def _sc_gather(table, y):
    mesh = plsc.VectorSubcoreMesh(core_axis_name="c", subcore_axis_name="s")
    return pl.kernel(
        _sc_body,
        out_type=jax.ShapeDtypeStruct((N_POINTS,), jnp.float32),
        mesh=mesh,
        compiler_params=pltpu.CompilerParams(needs_layout_passes=False),
        scratch_types=[
            pltpu.VMEM((N_BUCKETS,), jnp.float32),
            pltpu.VMEM((CHUNK,), jnp.float32),
            pltpu.VMEM((CHUNK,), jnp.float32),
        ],
    )(table, y)


def kernel(logits, y, boundaries):
    table = _build_table(logits, boundaries)
    return _sc_gather(table, y)


# speedup vs baseline: 18108.8809x; 1.4935x over previous
import functools

import jax
import jax.numpy as jnp
from jax import lax
from jax.experimental import pallas as pl
from jax.experimental.pallas import tpu as pltpu
from jax.experimental.pallas import tpu_sc as plsc

N_BUCKETS = 1024
N_POINTS = 8388608

NC = 2
NS = 16
NW = NC * NS
LANES = 16

PPW = N_POINTS // NW
CHUNK = 16384
N_CHUNKS = PPW // CHUNK


def _table_body(logits_ref, lo_ref, hi_ref, out_ref):
    l = logits_ref[...]
    m = jnp.max(l)
    lse = jnp.log(jnp.sum(jnp.exp(l - m))) + m
    w = hi_ref[...] - lo_ref[...]
    out_ref[...] = (l - lse) - jnp.log(w)


def _build_table(logits, boundaries):
    lo = boundaries[:-1].reshape(8, 128)
    hi = boundaries[1:].reshape(8, 128)
    table = pl.pallas_call(
        _table_body,
        out_shape=jax.ShapeDtypeStruct((8, 128), jnp.float32),
    )(logits.reshape(8, 128), lo, hi)
    return table.reshape(N_BUCKETS)


def _sc_body(table_hbm, y_hbm, out_hbm, table_v,
             y_v0, y_v1, out_v0, out_v1,
             sin0, sin1, sout0, sout1):
    wid = lax.axis_index("s") * NC + lax.axis_index("c")
    base = wid * PPW
    pltpu.sync_copy(table_hbm, table_v)

    y_bufs = (y_v0, y_v1)
    out_bufs = (out_v0, out_v1)
    sin = (sin0, sin1)
    sout = (sout0, sout1)

    def start_in(c):
        off = base + c * CHUNK
        return pltpu.async_copy(y_hbm.at[pl.ds(off, CHUNK)],
                                y_bufs[c % 2], sin[c % 2])

    def start_out(c):
        off = base + c * CHUNK
        return pltpu.async_copy(out_bufs[c % 2],
                                out_hbm.at[pl.ds(off, CHUNK)], sout[c % 2])

    in_h = {0: start_in(0)}
    out_h = {}
    for c in range(N_CHUNKS):
        b = c % 2
        if c + 1 < N_CHUNKS:
            in_h[c + 1] = start_in(c + 1)
        in_h.pop(c).wait()
        if c >= 2:
            out_h.pop(c - 2).wait()
        y_v = y_bufs[b]
        out_v = out_bufs[b]

        @plsc.parallel_loop(0, CHUNK // LANES, unroll=8)
        def grp_body(i):
            s = i * LANES
            y16 = y_v[pl.ds(s, LANES)]
            idx = jnp.minimum((y16 * float(N_BUCKETS)).astype(jnp.int32),
                              N_BUCKETS - 1)
            out_v[pl.ds(s, LANES)] = plsc.load_gather(table_v, [idx])

        out_h[c] = start_out(c)
    for c in list(out_h):
        out_h.pop(c).wait()


@jax.jit
def _sc_gather(table, y):
    mesh = plsc.VectorSubcoreMesh(core_axis_name="c", subcore_axis_name="s")
    return pl.kernel(
        _sc_body,
        out_type=jax.ShapeDtypeStruct((N_POINTS,), jnp.float32),
        mesh=mesh,
        compiler_params=pltpu.CompilerParams(needs_layout_passes=False),
        scratch_types=[
            pltpu.VMEM((N_BUCKETS,), jnp.float32),
            pltpu.VMEM((CHUNK,), jnp.float32),
            pltpu.VMEM((CHUNK,), jnp.float32),
            pltpu.VMEM((CHUNK,), jnp.float32),
            pltpu.VMEM((CHUNK,), jnp.float32),
            pltpu.SemaphoreType.DMA,
            pltpu.SemaphoreType.DMA,
            pltpu.SemaphoreType.DMA,
            pltpu.SemaphoreType.DMA,
        ],
    )(table, y)


def kernel(logits, y, boundaries):
    table = _build_table(logits, boundaries)
    return _sc_gather(table, y)
